# Initial kernel scaffold; baseline (speedup 1.0000x reference)
import jax
import jax.numpy as jnp
from jax.experimental import pallas as pl


def kernel(x, router, router_bias, w_gate_up, w_down):
    raise NotImplementedError("write your pallas kernel here")


# speedup vs baseline: 1.1535x; 1.1535x over previous
import functools

import jax
import jax.numpy as jnp
from jax.experimental import pallas as pl
from jax.experimental.pallas import tpu as pltpu

_NE = 8
_K = 2
_D = 1024
_I = 2816
_IJ = 256
_NJ = _I // _IJ


def _router_body(x_ref, r_ref, o_ref):
    o_ref[...] = jnp.dot(x_ref[...], r_ref[...],
                         preferred_element_type=jnp.float32)


def _mlp_body(wtok_ref, x_ref, wg_ref, wu_ref, wd_ref, o_ref):
    e = pl.program_id(0)
    j = pl.program_id(1)
    xg = x_ref[...]
    g = jnp.dot(xg, wg_ref[0], preferred_element_type=jnp.float32)
    u = jnp.dot(xg, wu_ref[0], preferred_element_type=jnp.float32)
    h = g * jax.nn.sigmoid(g) * u
    o = jnp.dot(h, wd_ref[0], preferred_element_type=jnp.float32)
    lane = jax.lax.broadcasted_iota(jnp.int32, (1, 128), 1)
    onehot = (lane == e).astype(jnp.float32)
    w = jnp.sum(wtok_ref[...] * onehot, axis=1, keepdims=True)
    contrib = w * o

    @pl.when((e == 0) & (j == 0))
    def _():
        o_ref[...] = contrib

    @pl.when(~((e == 0) & (j == 0)))
    def _():
        o_ref[...] += contrib


def kernel(x, router, router_bias, w_gate_up, w_down):
    b, s, d = x.shape
    t = b * s
    x_flat = x.reshape(t, d)

    router_p = jnp.zeros((d, 128), jnp.float32).at[:, :_NE].set(router)
    logits_p = pl.pallas_call(
        _router_body,
        out_shape=jax.ShapeDtypeStruct((t, 128), jnp.float32),
    )(x_flat, router_p)
    router_logits = logits_p[:, :_NE]

    biased_logits = router_logits + jax.lax.stop_gradient(router_bias)
    router_probs = jax.nn.softmax(router_logits, axis=-1)
    topk_logits, selected = jax.lax.top_k(biased_logits, _K + 1)
    qb_alpha = topk_logits[:, -1:]
    selected = selected[:, :-1]
    unbiased_topk = jnp.take_along_axis(router_logits, selected, axis=-1)
    combine_weights = jax.nn.sigmoid(unbiased_topk).astype(x.dtype)
    sel_onehot = jax.nn.one_hot(selected, _NE, dtype=jnp.float32)
    expert_counts = jnp.sum(sel_onehot, axis=(0, 1))
    total = jnp.maximum(jnp.sum(expert_counts), 1.0)
    frac = expert_counts / total
    routing_entropy = -jnp.sum(frac * jnp.log(frac + 1e-06))
    token_fraction = frac * _K
    p = jnp.mean(router_probs, axis=0)
    load_balancing_loss = _NE * jnp.sum(token_fraction * p)
    z = jax.nn.logsumexp(router_logits, axis=-1)
    router_z_loss = jnp.mean(z ** 2)
    s_minus_alpha = router_logits - qb_alpha
    qb_count = max(1, t * _K // _NE)
    topv, _ = jax.lax.top_k(s_minus_alpha.T, qb_count)
    qb_beta = topv[:, -1]

    wtok = jnp.sum(sel_onehot * combine_weights[..., None], axis=1)
    wtok_p = jnp.zeros((t, 128), jnp.float32).at[:, :_NE].set(wtok)

    wg = w_gate_up[:, :, :_I]
    wu = w_gate_up[:, :, _I:]
    routed = pl.pallas_call(
        _mlp_body,
        grid=(_NE, _NJ),
        in_specs=[
            pl.BlockSpec((t, 128), lambda e, j: (0, 0)),
            pl.BlockSpec((t, d), lambda e, j: (0, 0)),
            pl.BlockSpec((1, _D, _IJ), lambda e, j: (e, 0, j)),
            pl.BlockSpec((1, _D, _IJ), lambda e, j: (e, 0, j)),
            pl.BlockSpec((1, _IJ, _D), lambda e, j: (e, j, 0)),
        ],
        out_specs=pl.BlockSpec((t, d), lambda e, j: (0, 0)),
        out_shape=jax.ShapeDtypeStruct((t, d), jnp.float32),
    )(wtok_p, x_flat, wg, wu, w_down)

    return (routed.reshape(b, s, d), load_balancing_loss, router_z_loss,
            routing_entropy, expert_counts, qb_beta)
